# initial kernel scaffold (unmeasured)
import jax
import jax.numpy as jnp
from jax import lax
from jax.experimental import pallas as pl
from jax.experimental.pallas import tpu as pltpu

T = 2048
D = 4096
V = 16384
VS = V // 2

COL_BLK = 512
ROW_BLK = 128


def _cast_body(x_ref, o_ref):
    o_ref[...] = x_ref[...].astype(jnp.bfloat16)


def _matmul_body(xb_ref, w_ref, o_ref):
    acc = jnp.dot(
        xb_ref[...],
        w_ref[...].astype(jnp.bfloat16),
        preferred_element_type=jnp.float32,
    )
    o_ref[...] = acc.astype(jnp.bfloat16)


def _exchange_body(src_ref, dst_ref, send_sem, recv_sem):
    my_x = lax.axis_index("x")
    my_y = lax.axis_index("y")
    my_z = lax.axis_index("z")
    partner = (1 - my_x, my_y, my_z)

    barrier = pltpu.get_barrier_semaphore()
    pl.semaphore_signal(
        barrier, inc=1, device_id=partner,
        device_id_type=pl.DeviceIdType.MESH,
    )
    pl.semaphore_wait(barrier, 1)

    rdma = pltpu.make_async_remote_copy(
        src_ref=src_ref,
        dst_ref=dst_ref,
        send_sem=send_sem,
        recv_sem=recv_sem,
        device_id=partner,
        device_id_type=pl.DeviceIdType.MESH,
    )
    rdma.start()
    rdma.wait()


def _softmax_body(own_ref, oth_ref, o_ref):
    my_x = lax.axis_index("x")
    own = own_ref[...].astype(jnp.float32)
    oth = oth_ref[...].astype(jnp.float32)
    m = jnp.maximum(
        own.max(axis=-1, keepdims=True), oth.max(axis=-1, keepdims=True)
    )
    eo = jnp.exp(own - m)
    et = jnp.exp(oth - m)
    inv = 1.0 / (
        eo.sum(axis=-1, keepdims=True) + et.sum(axis=-1, keepdims=True)
    )
    o_ref[:, pl.ds(my_x * VS, VS)] = eo * inv
    o_ref[:, pl.ds((1 - my_x) * VS, VS)] = et * inv


def kernel(x, W):
    x_bf = pl.pallas_call(
        _cast_body,
        grid=(T // ROW_BLK,),
        in_specs=[pl.BlockSpec((ROW_BLK, D), lambda i: (i, 0))],
        out_specs=pl.BlockSpec((ROW_BLK, D), lambda i: (i, 0)),
        out_shape=jax.ShapeDtypeStruct((T, D), jnp.bfloat16),
    )(x)

    logits = pl.pallas_call(
        _matmul_body,
        grid=(VS // COL_BLK,),
        in_specs=[
            pl.BlockSpec((T, D), lambda j: (0, 0)),
            pl.BlockSpec((D, COL_BLK), lambda j: (0, j)),
        ],
        out_specs=pl.BlockSpec((T, COL_BLK), lambda j: (0, j)),
        out_shape=jax.ShapeDtypeStruct((T, VS), jnp.bfloat16),
    )(x_bf, W)

    oth = pl.pallas_call(
        _exchange_body,
        in_specs=[pl.BlockSpec(memory_space=pltpu.MemorySpace.ANY)],
        out_specs=pl.BlockSpec(memory_space=pltpu.MemorySpace.ANY),
        out_shape=jax.ShapeDtypeStruct((T, VS), jnp.bfloat16),
        scratch_shapes=[
            pltpu.SemaphoreType.DMA,
            pltpu.SemaphoreType.DMA,
        ],
        compiler_params=pltpu.CompilerParams(collective_id=0),
    )(logits)

    out = pl.pallas_call(
        _softmax_body,
        grid=(T // ROW_BLK,),
        in_specs=[
            pl.BlockSpec((ROW_BLK, VS), lambda i: (i, 0)),
            pl.BlockSpec((ROW_BLK, VS), lambda i: (i, 0)),
        ],
        out_specs=pl.BlockSpec((ROW_BLK, V), lambda i: (i, 0)),
        out_shape=jax.ShapeDtypeStruct((T, V), jnp.float32),
    )(logits, oth)
    return out


# baseline (device time: 599492 ns/iter reference)
import jax
import jax.numpy as jnp
from jax import lax
from jax.experimental import pallas as pl
from jax.experimental.pallas import tpu as pltpu

T = 2048
D = 4096
V = 16384
VS = V // 2

COL_BLK = 512
ROW_BLK = 128


def _cast_body(x_ref, o_ref):
    o_ref[...] = x_ref[...].astype(jnp.bfloat16)


def _matmul_body(xb_ref, w_ref, o_ref):
    acc = jnp.dot(
        xb_ref[...],
        w_ref[...].astype(jnp.bfloat16),
        preferred_element_type=jnp.float32,
    )
    o_ref[...] = acc.astype(jnp.bfloat16)


def _exchange_body(src_ref, dst_ref, send_sem, recv_sem):
    my_x = lax.axis_index("x")
    my_y = lax.axis_index("y")
    my_z = lax.axis_index("z")
    partner = (1 - my_x, my_y, my_z)

    barrier = pltpu.get_barrier_semaphore()
    pl.semaphore_signal(
        barrier, inc=1, device_id=partner,
        device_id_type=pl.DeviceIdType.MESH,
    )
    pl.semaphore_wait(barrier, 1)

    rdma = pltpu.make_async_remote_copy(
        src_ref=src_ref,
        dst_ref=dst_ref,
        send_sem=send_sem,
        recv_sem=recv_sem,
        device_id=partner,
        device_id_type=pl.DeviceIdType.MESH,
    )
    rdma.start()
    rdma.wait()


def _softmax_body(own_ref, oth_ref, o_ref):
    my_x = lax.axis_index("x")
    own = own_ref[...].astype(jnp.float32)
    oth = oth_ref[...].astype(jnp.float32)
    m = jnp.maximum(
        own.max(axis=-1, keepdims=True), oth.max(axis=-1, keepdims=True)
    )
    eo = jnp.exp(own - m)
    et = jnp.exp(oth - m)
    inv = 1.0 / (
        eo.sum(axis=-1, keepdims=True) + et.sum(axis=-1, keepdims=True)
    )
    o_ref[:, pl.ds(my_x * VS, VS)] = eo * inv
    o_ref[:, pl.ds((1 - my_x) * VS, VS)] = et * inv


def kernel(x, W):
    x_bf = pl.pallas_call(
        _cast_body,
        grid=(T // ROW_BLK,),
        in_specs=[pl.BlockSpec((ROW_BLK, D), lambda i: (i, 0))],
        out_specs=pl.BlockSpec((ROW_BLK, D), lambda i: (i, 0)),
        out_shape=jax.ShapeDtypeStruct((T, D), jnp.bfloat16),
    )(x)

    logits = pl.pallas_call(
        _matmul_body,
        grid=(VS // COL_BLK,),
        in_specs=[
            pl.BlockSpec((T, D), lambda j: (0, 0)),
            pl.BlockSpec((D, COL_BLK), lambda j: (0, j)),
        ],
        out_specs=pl.BlockSpec((T, COL_BLK), lambda j: (0, j)),
        out_shape=jax.ShapeDtypeStruct((T, VS), jnp.bfloat16),
    )(x_bf, W)

    oth = pl.pallas_call(
        _exchange_body,
        in_specs=[pl.BlockSpec(memory_space=pl.ANY)],
        out_specs=pl.BlockSpec(memory_space=pl.ANY),
        out_shape=jax.ShapeDtypeStruct((T, VS), jnp.bfloat16),
        scratch_shapes=[
            pltpu.SemaphoreType.DMA,
            pltpu.SemaphoreType.DMA,
        ],
        compiler_params=pltpu.CompilerParams(collective_id=0),
    )(logits)

    out = pl.pallas_call(
        _softmax_body,
        grid=(T // ROW_BLK,),
        in_specs=[
            pl.BlockSpec((ROW_BLK, VS), lambda i: (i, 0)),
            pl.BlockSpec((ROW_BLK, VS), lambda i: (i, 0)),
        ],
        out_specs=pl.BlockSpec((ROW_BLK, V), lambda i: (i, 0)),
        out_shape=jax.ShapeDtypeStruct((T, V), jnp.float32),
    )(logits, oth)
    return out


# device time: 462510 ns/iter; 1.2962x vs baseline; 1.2962x over previous
import jax
import jax.numpy as jnp
from jax import lax
from jax.experimental import pallas as pl
from jax.experimental.pallas import tpu as pltpu

T = 2048
D = 4096
V = 16384
VS = V // 2

COL_BLK = 512
N_CHUNK = VS // COL_BLK
SLOTS = 2
ROW_BLK = 128


def _cast_body(x_ref, o_ref):
    o_ref[...] = x_ref[...].astype(jnp.bfloat16)


def _mm_exchange_body(
    xb_ref, w_ref, own_ref, oth_ref, comm_ref, send_sems, recv_sems
):
    j = pl.program_id(0)
    my_x = lax.axis_index("x")
    my_y = lax.axis_index("y")
    my_z = lax.axis_index("z")
    partner = (1 - my_x, my_y, my_z)

    @pl.when(j == 0)
    def _():
        barrier = pltpu.get_barrier_semaphore()
        pl.semaphore_signal(
            barrier, inc=1, device_id=partner,
            device_id_type=pl.DeviceIdType.MESH,
        )
        pl.semaphore_wait(barrier, 1)

    acc = jnp.dot(
        xb_ref[...],
        w_ref[...].astype(jnp.bfloat16),
        preferred_element_type=jnp.float32,
    ).astype(jnp.bfloat16)
    own_ref[...] = acc

    slot = j % SLOTS

    @pl.when(j >= SLOTS)
    def _():
        pltpu.make_async_remote_copy(
            src_ref=comm_ref.at[slot],
            dst_ref=oth_ref.at[:, pl.ds((j - SLOTS) * COL_BLK, COL_BLK)],
            send_sem=send_sems.at[slot],
            recv_sem=recv_sems.at[j - SLOTS],
            device_id=partner,
            device_id_type=pl.DeviceIdType.MESH,
        ).wait_send()

    comm_ref[slot] = acc
    pltpu.make_async_remote_copy(
        src_ref=comm_ref.at[slot],
        dst_ref=oth_ref.at[:, pl.ds(j * COL_BLK, COL_BLK)],
        send_sem=send_sems.at[slot],
        recv_sem=recv_sems.at[j],
        device_id=partner,
        device_id_type=pl.DeviceIdType.MESH,
    ).start()

    @pl.when(j == N_CHUNK - 1)
    def _():
        for i in range(N_CHUNK - SLOTS, N_CHUNK):
            pltpu.make_async_remote_copy(
                src_ref=comm_ref.at[i % SLOTS],
                dst_ref=oth_ref.at[:, pl.ds(i * COL_BLK, COL_BLK)],
                send_sem=send_sems.at[i % SLOTS],
                recv_sem=recv_sems.at[i],
                device_id=partner,
                device_id_type=pl.DeviceIdType.MESH,
            ).wait_send()
        for i in range(N_CHUNK):
            pltpu.make_async_remote_copy(
                src_ref=comm_ref.at[i % SLOTS],
                dst_ref=oth_ref.at[:, pl.ds(i * COL_BLK, COL_BLK)],
                send_sem=send_sems.at[i % SLOTS],
                recv_sem=recv_sems.at[i],
                device_id=partner,
                device_id_type=pl.DeviceIdType.MESH,
            ).wait_recv()


def _softmax_body(own_ref, oth_ref, o_ref):
    my_x = lax.axis_index("x")
    own = own_ref[...].astype(jnp.float32)
    oth = oth_ref[...].astype(jnp.float32)
    m = jnp.maximum(
        own.max(axis=-1, keepdims=True), oth.max(axis=-1, keepdims=True)
    )
    eo = jnp.exp(own - m)
    et = jnp.exp(oth - m)
    inv = 1.0 / (
        eo.sum(axis=-1, keepdims=True) + et.sum(axis=-1, keepdims=True)
    )
    o_ref[:, pl.ds(my_x * VS, VS)] = eo * inv
    o_ref[:, pl.ds((1 - my_x) * VS, VS)] = et * inv


def kernel(x, W):
    x_bf = pl.pallas_call(
        _cast_body,
        grid=(T // ROW_BLK,),
        in_specs=[pl.BlockSpec((ROW_BLK, D), lambda i: (i, 0))],
        out_specs=pl.BlockSpec((ROW_BLK, D), lambda i: (i, 0)),
        out_shape=jax.ShapeDtypeStruct((T, D), jnp.bfloat16),
    )(x)

    logits, oth = pl.pallas_call(
        _mm_exchange_body,
        grid=(N_CHUNK,),
        in_specs=[
            pl.BlockSpec((T, D), lambda j: (0, 0)),
            pl.BlockSpec((D, COL_BLK), lambda j: (0, j)),
        ],
        out_specs=[
            pl.BlockSpec((T, COL_BLK), lambda j: (0, j)),
            pl.BlockSpec(memory_space=pl.ANY),
        ],
        out_shape=[
            jax.ShapeDtypeStruct((T, VS), jnp.bfloat16),
            jax.ShapeDtypeStruct((T, VS), jnp.bfloat16),
        ],
        scratch_shapes=[
            pltpu.VMEM((SLOTS, T, COL_BLK), jnp.bfloat16),
            pltpu.SemaphoreType.DMA((SLOTS,)),
            pltpu.SemaphoreType.DMA((N_CHUNK,)),
        ],
        compiler_params=pltpu.CompilerParams(
            collective_id=0,
            dimension_semantics=("arbitrary",),
        ),
    )(x_bf, W)

    out = pl.pallas_call(
        _softmax_body,
        grid=(T // ROW_BLK,),
        in_specs=[
            pl.BlockSpec((ROW_BLK, VS), lambda i: (i, 0)),
            pl.BlockSpec((ROW_BLK, VS), lambda i: (i, 0)),
        ],
        out_specs=pl.BlockSpec((ROW_BLK, V), lambda i: (i, 0)),
        out_shape=jax.ShapeDtypeStruct((T, V), jnp.float32),
    )(logits, oth)
    return out
